# baseline (speedup 1.0000x reference)
import functools

import jax
import jax.numpy as jnp
from jax import lax
from jax.experimental import pallas as pl
from jax.experimental.pallas import tpu as pltpu
from jax.experimental.pallas import tpu_sc as plsc

VOCAB = 1000000
D = 64
B = 16384
L = 50
NUM_CLASS = 5

NC = 2
NS = 16
NW = NC * NS
BAGS_PER_W = B // NW
BAGS_PER_CHUNK = 2
ROWS_PER_CHUNK = BAGS_PER_CHUNK * L
NCHUNK = BAGS_PER_W // BAGS_PER_CHUNK
NBUF = 8

VBLK = 8192
N_VBLK = -(-VOCAB // VBLK)
PIECE = VBLK // 8
PACK_ROWS = N_VBLK * (VBLK // 8)


def _tc_proj_body(x_ref, w_ref, b_ref, o_ref):
    y = lax.dot_general(x_ref[...].astype(jnp.bfloat16),
                        w_ref[...].astype(jnp.bfloat16),
                        (((0,), (0,)), ((), ())),
                        preferred_element_type=jnp.float32)
    y = y + b_ref[...]
    lane = lax.broadcasted_iota(jnp.int32, (PIECE, 128), 1)
    slot = lax.shift_right_logical(lane, 4)
    acc = jnp.zeros((PIECE, 128), jnp.float32)
    for k in range(8):
        acc = acc + jnp.where(slot == k, y[PIECE * k:PIECE * (k + 1), :], 0.0)
    o_ref[...] = acc


def _tc_proj(table_t, w_pad, b_pad):
    return pl.pallas_call(
        _tc_proj_body,
        grid=(N_VBLK,),
        in_specs=[
            pl.BlockSpec((D, VBLK), lambda i: (0, i)),
            pl.BlockSpec((D, 128), lambda i: (0, 0)),
            pl.BlockSpec((1, 128), lambda i: (0, 0)),
        ],
        out_specs=pl.BlockSpec((VBLK // 8, 128), lambda i: (i, 0)),
        out_shape=jax.ShapeDtypeStruct((PACK_ROWS, 128), jnp.float32),
        compiler_params=pltpu.CompilerParams(
            fuse_transposed_lhs_in_matmul=True),
    )(table_t, w_pad, b_pad)


def _sc_body(idx_hbm, proj_hbm, out_hbm, idx_v, gbuf, outb, sem0, sem1,
             sem2, sem3, sem4, sem5, sem6, sem7):
    wid = lax.axis_index("s") * NC + lax.axis_index("c")
    row0 = wid * NCHUNK
    pltpu.sync_copy(idx_hbm.at[pl.ds(row0, NCHUNK)], idx_v)

    sems = (sem0, sem1, sem2, sem3, sem4, sem5, sem6, sem7)

    def start(j, b):
        pltpu.async_copy(proj_hbm.at[idx_v.at[j]], gbuf.at[b], sems[b])

    def wait(b):
        pltpu.make_async_copy(proj_hbm.at[idx_v.at[0]], gbuf.at[b], sems[b]).wait()

    for b in range(NBUF):
        start(b, b)

    zeros = jnp.zeros((16,), jnp.float32)
    iota16 = lax.iota(jnp.int32, 16)
    lo_mask = iota16 < 8
    perm_idx = lax.rem(iota16, 8)

    @pl.loop(0, NCHUNK, step=NBUF)
    def _chunks(g):
        for b in range(NBUF):
            j = g + b
            wait(b)

            accs = []
            for bic in range(BAGS_PER_CHUNK):
                r0 = bic * L

                def body(r, acc):
                    return acc + gbuf[b, r0 + r, pl.ds(0, 16)]

                accs.append(lax.fori_loop(0, L, body, zeros, unroll=4))

            hi = lax.gather(
                accs[1], perm_idx[:, None],
                lax.GatherDimensionNumbers(offset_dims=(),
                                           collapsed_slice_dims=(0,),
                                           start_index_map=(0,)),
                slice_sizes=(1,),
                mode=lax.GatherScatterMode.PROMISE_IN_BOUNDS)
            combined = jnp.where(lo_mask, accs[0], hi)
            outb[lax.shift_right_logical(j, 3),
                 pl.ds(lax.bitwise_and(j, 7) * 16, 16)] = combined

            @pl.when(j + NBUF < NCHUNK)
            def _():
                start(j + NBUF, b)

    pltpu.sync_copy(outb, out_hbm.at[pl.ds(wid * (BAGS_PER_W // 16),
                                           BAGS_PER_W // 16)])


@functools.partial(
    pl.kernel,
    out_type=jax.ShapeDtypeStruct((B // 16, 128), jnp.float32),
    mesh=plsc.VectorSubcoreMesh(core_axis_name="c", subcore_axis_name="s",
                                num_cores=NC, num_subcores=NS),
    scratch_types=[
        pltpu.VMEM((NCHUNK, ROWS_PER_CHUNK), jnp.int32),
        pltpu.VMEM((NBUF, ROWS_PER_CHUNK, 16), jnp.float32),
        pltpu.VMEM((BAGS_PER_W // 16, 128), jnp.float32),
        pltpu.SemaphoreType.DMA,
        pltpu.SemaphoreType.DMA,
        pltpu.SemaphoreType.DMA,
        pltpu.SemaphoreType.DMA,
        pltpu.SemaphoreType.DMA,
        pltpu.SemaphoreType.DMA,
        pltpu.SemaphoreType.DMA,
        pltpu.SemaphoreType.DMA,
    ],
    compiler_params=pltpu.CompilerParams(use_tc_tiling_on_sc=False),
)
def _sc_bag_sums(idx_hbm, proj_hbm, out_hbm, idx_v, gbuf, outb, sem0, sem1,
                 sem2, sem3, sem4, sem5, sem6, sem7):
    _sc_body(idx_hbm, proj_hbm, out_hbm, idx_v, gbuf, outb, sem0, sem1,
             sem2, sem3, sem4, sem5, sem6, sem7)


def kernel(text, emb_table, fc_w, fc_b):
    t32 = text.astype(jnp.int32)
    hidx = (
        lax.bitwise_and(t32, -VBLK)
        + lax.bitwise_and(t32, PIECE - 1) * 8
        + lax.shift_right_logical(lax.bitwise_and(t32, VBLK - 1), 10)
    )
    hidx2d = hidx.reshape(NW * NCHUNK, ROWS_PER_CHUNK)

    w16 = jnp.zeros((D, 16), jnp.float32)
    w16 = w16.at[:, :NUM_CLASS].set(fc_w.T / L)
    w16 = w16.at[:, 8:8 + NUM_CLASS].set(fc_w.T / L)
    w_pad = jnp.tile(w16, (1, 8))
    b16 = jnp.zeros((1, 16), jnp.float32)
    b16 = b16.at[0, :NUM_CLASS].set(fc_b / L)
    b16 = b16.at[0, 8:8 + NUM_CLASS].set(fc_b / L)
    b_pad = jnp.tile(b16, (1, 8))

    proj = _tc_proj(emb_table.T, w_pad, b_pad)
    proj16 = proj.reshape(PACK_ROWS * 8, 16)

    packed = _sc_bag_sums(hidx2d, proj16)
    return packed.reshape(B, 8)[:, :NUM_CLASS]


# speedup vs baseline: 1.2730x; 1.2730x over previous
import functools

import jax
import jax.numpy as jnp
from jax import lax
from jax.experimental import pallas as pl
from jax.experimental.pallas import tpu as pltpu
from jax.experimental.pallas import tpu_sc as plsc

VOCAB = 1000000
D = 64
B = 16384
L = 50
NUM_CLASS = 5

NC = 2
NS = 16
NW = NC * NS
BAGS_PER_W = B // NW
BAGS_PER_CHUNK = 2
ROWS_PER_CHUNK = BAGS_PER_CHUNK * L
NCHUNK = BAGS_PER_W // BAGS_PER_CHUNK
NBUF = 8

VBLK = 32768
N_VBLK = -(-VOCAB // VBLK)
PIECE = VBLK // 8
PACK_ROWS = N_VBLK * (VBLK // 8)


def _tc_proj_body(x_ref, w_ref, b_ref, o_ref):
    y = lax.dot_general(x_ref[...].astype(jnp.bfloat16),
                        w_ref[...].astype(jnp.bfloat16),
                        (((0,), (0,)), ((), ())),
                        preferred_element_type=jnp.float32)
    y = y + b_ref[...]
    lane = lax.broadcasted_iota(jnp.int32, (PIECE, 128), 1)
    slot = lax.shift_right_logical(lane, 4)
    acc = jnp.zeros((PIECE, 128), jnp.float32)
    for k in range(8):
        acc = acc + jnp.where(slot == k, y[PIECE * k:PIECE * (k + 1), :], 0.0)
    o_ref[...] = acc


def _tc_proj(table_t, w_pad, b_pad):
    return pl.pallas_call(
        _tc_proj_body,
        grid=(N_VBLK,),
        in_specs=[
            pl.BlockSpec((D, VBLK), lambda i: (0, i)),
            pl.BlockSpec((D, 128), lambda i: (0, 0)),
            pl.BlockSpec((1, 128), lambda i: (0, 0)),
        ],
        out_specs=pl.BlockSpec((VBLK // 8, 128), lambda i: (i, 0)),
        out_shape=jax.ShapeDtypeStruct((PACK_ROWS, 128), jnp.float32),
        compiler_params=pltpu.CompilerParams(
            fuse_transposed_lhs_in_matmul=True,
            vmem_limit_bytes=110 * 1024 * 1024),
    )(table_t, w_pad, b_pad)


def _sc_body(idx_hbm, proj_hbm, out_hbm, idx_v, gbuf, outb, sem0, sem1,
             sem2, sem3, sem4, sem5, sem6, sem7):
    wid = lax.axis_index("s") * NC + lax.axis_index("c")
    row0 = wid * NCHUNK
    pltpu.sync_copy(idx_hbm.at[pl.ds(row0, NCHUNK)], idx_v)

    sems = (sem0, sem1, sem2, sem3, sem4, sem5, sem6, sem7)

    def start(j, b):
        pltpu.async_copy(proj_hbm.at[idx_v.at[j]], gbuf.at[b], sems[b])

    def wait(b):
        pltpu.make_async_copy(proj_hbm.at[idx_v.at[0]], gbuf.at[b], sems[b]).wait()

    for b in range(NBUF):
        start(b, b)

    zeros = jnp.zeros((16,), jnp.float32)
    iota16 = lax.iota(jnp.int32, 16)
    lo_mask = iota16 < 8
    perm_idx = lax.rem(iota16, 8)

    @pl.loop(0, NCHUNK, step=NBUF)
    def _chunks(g):
        for b in range(NBUF):
            j = g + b
            wait(b)

            accs = []
            for bic in range(BAGS_PER_CHUNK):
                r0 = bic * L

                def body(r, acc):
                    return acc + gbuf[b, r0 + r, pl.ds(0, 16)]

                accs.append(lax.fori_loop(0, L, body, zeros, unroll=4))

            hi = lax.gather(
                accs[1], perm_idx[:, None],
                lax.GatherDimensionNumbers(offset_dims=(),
                                           collapsed_slice_dims=(0,),
                                           start_index_map=(0,)),
                slice_sizes=(1,),
                mode=lax.GatherScatterMode.PROMISE_IN_BOUNDS)
            combined = jnp.where(lo_mask, accs[0], hi)
            outb[lax.shift_right_logical(j, 3),
                 pl.ds(lax.bitwise_and(j, 7) * 16, 16)] = combined

            @pl.when(j + NBUF < NCHUNK)
            def _():
                start(j + NBUF, b)

    pltpu.sync_copy(outb, out_hbm.at[pl.ds(wid * (BAGS_PER_W // 16),
                                           BAGS_PER_W // 16)])


@functools.partial(
    pl.kernel,
    out_type=jax.ShapeDtypeStruct((B // 16, 128), jnp.float32),
    mesh=plsc.VectorSubcoreMesh(core_axis_name="c", subcore_axis_name="s",
                                num_cores=NC, num_subcores=NS),
    scratch_types=[
        pltpu.VMEM((NCHUNK, ROWS_PER_CHUNK), jnp.int32),
        pltpu.VMEM((NBUF, ROWS_PER_CHUNK, 16), jnp.float32),
        pltpu.VMEM((BAGS_PER_W // 16, 128), jnp.float32),
        pltpu.SemaphoreType.DMA,
        pltpu.SemaphoreType.DMA,
        pltpu.SemaphoreType.DMA,
        pltpu.SemaphoreType.DMA,
        pltpu.SemaphoreType.DMA,
        pltpu.SemaphoreType.DMA,
        pltpu.SemaphoreType.DMA,
        pltpu.SemaphoreType.DMA,
    ],
    compiler_params=pltpu.CompilerParams(use_tc_tiling_on_sc=False),
)
def _sc_bag_sums(idx_hbm, proj_hbm, out_hbm, idx_v, gbuf, outb, sem0, sem1,
                 sem2, sem3, sem4, sem5, sem6, sem7):
    _sc_body(idx_hbm, proj_hbm, out_hbm, idx_v, gbuf, outb, sem0, sem1,
             sem2, sem3, sem4, sem5, sem6, sem7)


def kernel(text, emb_table, fc_w, fc_b):
    t32 = text.astype(jnp.int32)
    hidx = (
        lax.bitwise_and(t32, -VBLK)
        + lax.bitwise_and(t32, PIECE - 1) * 8
        + lax.shift_right_logical(lax.bitwise_and(t32, VBLK - 1), 12)
    )
    hidx2d = hidx.reshape(NW * NCHUNK, ROWS_PER_CHUNK)

    w16 = jnp.zeros((D, 16), jnp.float32)
    w16 = w16.at[:, :NUM_CLASS].set(fc_w.T / L)
    w16 = w16.at[:, 8:8 + NUM_CLASS].set(fc_w.T / L)
    w_pad = jnp.tile(w16, (1, 8))
    b16 = jnp.zeros((1, 16), jnp.float32)
    b16 = b16.at[0, :NUM_CLASS].set(fc_b / L)
    b16 = b16.at[0, 8:8 + NUM_CLASS].set(fc_b / L)
    b_pad = jnp.tile(b16, (1, 8))

    proj = _tc_proj(emb_table.T, w_pad, b_pad)
    proj16 = proj.reshape(PACK_ROWS * 8, 16)

    packed = _sc_bag_sums(hidx2d, proj16)
    return packed.reshape(B, 8)[:, :NUM_CLASS]


# speedup vs baseline: 1.2801x; 1.0056x over previous
import functools

import jax
import jax.numpy as jnp
from jax import lax
from jax.experimental import pallas as pl
from jax.experimental.pallas import tpu as pltpu
from jax.experimental.pallas import tpu_sc as plsc

VOCAB = 1000000
D = 64
B = 16384
L = 50
NUM_CLASS = 5

NC = 2
NS = 16
NW = NC * NS
BAGS_PER_W = B // NW
BAGS_PER_CHUNK = 2
ROWS_PER_CHUNK = BAGS_PER_CHUNK * L
NCHUNK = BAGS_PER_W // BAGS_PER_CHUNK
NBUF = 8

VBLK = 65536
N_VBLK = -(-VOCAB // VBLK)
PIECE = VBLK // 8
PACK_ROWS = N_VBLK * (VBLK // 8)


def _tc_proj_body(x_ref, w_ref, b_ref, o_ref):
    y = lax.dot_general(x_ref[...].astype(jnp.bfloat16),
                        w_ref[...].astype(jnp.bfloat16),
                        (((0,), (0,)), ((), ())),
                        preferred_element_type=jnp.float32)
    y = y + b_ref[...]
    lane = lax.broadcasted_iota(jnp.int32, (PIECE, 128), 1)
    slot = lax.shift_right_logical(lane, 4)
    acc = jnp.zeros((PIECE, 128), jnp.float32)
    for k in range(8):
        acc = acc + jnp.where(slot == k, y[PIECE * k:PIECE * (k + 1), :], 0.0)
    o_ref[...] = acc


def _tc_proj(table_t, w_pad, b_pad):
    return pl.pallas_call(
        _tc_proj_body,
        grid=(N_VBLK,),
        in_specs=[
            pl.BlockSpec((D, VBLK), lambda i: (0, i)),
            pl.BlockSpec((D, 128), lambda i: (0, 0)),
            pl.BlockSpec((1, 128), lambda i: (0, 0)),
        ],
        out_specs=pl.BlockSpec((VBLK // 8, 128), lambda i: (i, 0)),
        out_shape=jax.ShapeDtypeStruct((PACK_ROWS, 128), jnp.float32),
        compiler_params=pltpu.CompilerParams(
            fuse_transposed_lhs_in_matmul=True,
            vmem_limit_bytes=110 * 1024 * 1024),
    )(table_t, w_pad, b_pad)


def _sc_body(idx_hbm, proj_hbm, out_hbm, idx_v, gbuf, outb, sem0, sem1,
             sem2, sem3, sem4, sem5, sem6, sem7):
    wid = lax.axis_index("s") * NC + lax.axis_index("c")
    row0 = wid * NCHUNK
    pltpu.sync_copy(idx_hbm.at[pl.ds(row0, NCHUNK)], idx_v)

    sems = (sem0, sem1, sem2, sem3, sem4, sem5, sem6, sem7)

    def start(j, b):
        pltpu.async_copy(proj_hbm.at[idx_v.at[j]], gbuf.at[b], sems[b])

    def wait(b):
        pltpu.make_async_copy(proj_hbm.at[idx_v.at[0]], gbuf.at[b], sems[b]).wait()

    for b in range(NBUF):
        start(b, b)

    zeros = jnp.zeros((16,), jnp.float32)
    iota16 = lax.iota(jnp.int32, 16)
    lo_mask = iota16 < 8
    perm_idx = lax.rem(iota16, 8)

    @pl.loop(0, NCHUNK, step=NBUF)
    def _chunks(g):
        for b in range(NBUF):
            j = g + b
            wait(b)

            accs = []
            for bic in range(BAGS_PER_CHUNK):
                r0 = bic * L

                def body(r, acc):
                    return acc + gbuf[b, r0 + r, pl.ds(0, 16)]

                accs.append(lax.fori_loop(0, L, body, zeros, unroll=4))

            hi = lax.gather(
                accs[1], perm_idx[:, None],
                lax.GatherDimensionNumbers(offset_dims=(),
                                           collapsed_slice_dims=(0,),
                                           start_index_map=(0,)),
                slice_sizes=(1,),
                mode=lax.GatherScatterMode.PROMISE_IN_BOUNDS)
            combined = jnp.where(lo_mask, accs[0], hi)
            outb[lax.shift_right_logical(j, 3),
                 pl.ds(lax.bitwise_and(j, 7) * 16, 16)] = combined

            @pl.when(j + NBUF < NCHUNK)
            def _():
                start(j + NBUF, b)

    pltpu.sync_copy(outb, out_hbm.at[pl.ds(wid * (BAGS_PER_W // 16),
                                           BAGS_PER_W // 16)])


@functools.partial(
    pl.kernel,
    out_type=jax.ShapeDtypeStruct((B // 16, 128), jnp.float32),
    mesh=plsc.VectorSubcoreMesh(core_axis_name="c", subcore_axis_name="s",
                                num_cores=NC, num_subcores=NS),
    scratch_types=[
        pltpu.VMEM((NCHUNK, ROWS_PER_CHUNK), jnp.int32),
        pltpu.VMEM((NBUF, ROWS_PER_CHUNK, 16), jnp.float32),
        pltpu.VMEM((BAGS_PER_W // 16, 128), jnp.float32),
        pltpu.SemaphoreType.DMA,
        pltpu.SemaphoreType.DMA,
        pltpu.SemaphoreType.DMA,
        pltpu.SemaphoreType.DMA,
        pltpu.SemaphoreType.DMA,
        pltpu.SemaphoreType.DMA,
        pltpu.SemaphoreType.DMA,
        pltpu.SemaphoreType.DMA,
    ],
    compiler_params=pltpu.CompilerParams(use_tc_tiling_on_sc=False),
)
def _sc_bag_sums(idx_hbm, proj_hbm, out_hbm, idx_v, gbuf, outb, sem0, sem1,
                 sem2, sem3, sem4, sem5, sem6, sem7):
    _sc_body(idx_hbm, proj_hbm, out_hbm, idx_v, gbuf, outb, sem0, sem1,
             sem2, sem3, sem4, sem5, sem6, sem7)


def kernel(text, emb_table, fc_w, fc_b):
    t32 = text.astype(jnp.int32)
    hidx = (
        lax.bitwise_and(t32, -VBLK)
        + lax.bitwise_and(t32, PIECE - 1) * 8
        + lax.shift_right_logical(lax.bitwise_and(t32, VBLK - 1), 13)
    )
    hidx2d = hidx.reshape(NW * NCHUNK, ROWS_PER_CHUNK)

    w16 = jnp.zeros((D, 16), jnp.float32)
    w16 = w16.at[:, :NUM_CLASS].set(fc_w.T / L)
    w16 = w16.at[:, 8:8 + NUM_CLASS].set(fc_w.T / L)
    w_pad = jnp.tile(w16, (1, 8))
    b16 = jnp.zeros((1, 16), jnp.float32)
    b16 = b16.at[0, :NUM_CLASS].set(fc_b / L)
    b16 = b16.at[0, 8:8 + NUM_CLASS].set(fc_b / L)
    b_pad = jnp.tile(b16, (1, 8))

    proj = _tc_proj(emb_table.T, w_pad, b_pad)
    proj16 = proj.reshape(PACK_ROWS * 8, 16)

    packed = _sc_bag_sums(hidx2d, proj16)
    return packed.reshape(B, 8)[:, :NUM_CLASS]
